# baseline (speedup 1.0000x reference)
import functools

import jax
import jax.numpy as jnp
from jax import lax
from jax.experimental import pallas as pl
from jax.experimental.pallas import tpu as pltpu
from jax.experimental.pallas import tpu_sc as plsc

B = 16384
KF = 64
NC = 2
NS = 16
NW = NC * NS
BPW = B // NW
NCH = 4
CH = BPW // NCH


def _time_table_kernel(dt_ref, wk_ref, yr_ref, w1_ref, b1_ref, w2_ref,
                       b2_ref, gb_ref, out_ref):
    row = lax.broadcasted_iota(jnp.int32, (128, 128), 0)
    col = lax.broadcasted_iota(jnp.int32, (128, 128), 1)
    d = row // 40
    w = (row % 40) // 20
    y = row % 20
    ohd = jnp.where(col == d, 1.0, 0.0)
    ohw = jnp.where(col == w, 1.0, 0.0)
    ohy = jnp.where(col == y, 1.0, 0.0)
    f = (jnp.dot(ohd, dt_ref[...], preferred_element_type=jnp.float32)
         + jnp.dot(ohw, wk_ref[...], preferred_element_type=jnp.float32)
         + jnp.dot(ohy, yr_ref[...], preferred_element_type=jnp.float32))
    h = jnp.maximum(
        jnp.dot(f, w1_ref[...], preferred_element_type=jnp.float32)
        + b1_ref[...], 0.0)
    e = jnp.dot(h, w2_ref[...], preferred_element_type=jnp.float32)
    bias = f[:, 30:31] + f[:, 31:32] + f[:, 32:33]
    out_ref[...] = e + bias + b2_ref[0, 0] + gb_ref[0, 0]


def _build_time_table(daytime_emb, weekend_emb, year_emb,
                      daytime_bias_w, weekend_bias_w, year_bias_w,
                      fc_w1, fc_b1, fc_w2, fc_b2, global_bias):
    f32 = jnp.float32
    z = jnp.zeros((128, 128), f32)
    dt = z.at[0:3, 0:10].set(daytime_emb).at[0:3, 30].set(daytime_bias_w[:, 0])
    wk = z.at[0:2, 10:20].set(weekend_emb).at[0:2, 31].set(weekend_bias_w[:, 0])
    yr = z.at[0:20, 20:30].set(year_emb).at[0:20, 32].set(year_bias_w[:, 0])
    w1 = z.at[0:30, 0:10].set(fc_w1.T)
    w2 = z.at[0:10, 0:1].set(fc_w2.T)
    b1 = jnp.zeros((1, 128), f32).at[0, 0:10].set(fc_b1)
    tmat = pl.pallas_call(
        _time_table_kernel,
        out_shape=jax.ShapeDtypeStruct((128, 128), f32),
        in_specs=[
            pl.BlockSpec(memory_space=pltpu.VMEM),
            pl.BlockSpec(memory_space=pltpu.VMEM),
            pl.BlockSpec(memory_space=pltpu.VMEM),
            pl.BlockSpec(memory_space=pltpu.VMEM),
            pl.BlockSpec(memory_space=pltpu.VMEM),
            pl.BlockSpec(memory_space=pltpu.VMEM),
            pl.BlockSpec(memory_space=pltpu.SMEM),
            pl.BlockSpec(memory_space=pltpu.SMEM),
        ],
        out_specs=pl.BlockSpec(memory_space=pltpu.VMEM),
    )(dt, wk, yr, w1, b1, w2,
      fc_b2.reshape(1, 1), global_bias.reshape(1, 1))
    return tmat[:, 0]


def _sc_body(uin_h, iin_h, din_h, win_h, yin_h, uembT_h, iembT_h, ubT_h,
             ibT_h, utbT_h, t_h, out_h,
             uidx_v, iidx_v, didx_v, widx_v, yidx_v, combo_v,
             U_v, I_v, ub_v, ib_v, utb0_v, utb1_v, utb2_v, t_v, res_v, sem):
    wid = lax.axis_index("s") * NC + lax.axis_index("c")
    base = wid * BPW
    pltpu.sync_copy(uin_h.at[pl.ds(base, BPW)], uidx_v)
    pltpu.sync_copy(iin_h.at[pl.ds(base, BPW)], iidx_v)
    pltpu.sync_copy(din_h.at[pl.ds(base, BPW)], didx_v)
    pltpu.sync_copy(win_h.at[pl.ds(base, BPW)], widx_v)
    pltpu.sync_copy(yin_h.at[pl.ds(base, BPW)], yidx_v)
    pltpu.sync_copy(t_h, t_v)

    for g in range(BPW // 16):
        sl = pl.ds(g * 16, 16)
        combo_v[sl] = didx_v[sl] * 40 + widx_v[sl] * 20 + yidx_v[sl]

    def chunk_copies(k):
        cps = []
        for j in range(NCH):
            isl = pl.ds(j * CH, CH)
            cps.append((uembT_h.at[k].at[uidx_v.at[isl]], U_v.at[k].at[isl]))
            cps.append((iembT_h.at[k].at[iidx_v.at[isl]], I_v.at[k].at[isl]))
        return cps

    def bias_copies():
        cps = []
        for j in range(NCH):
            isl = pl.ds(j * CH, CH)
            cps.append((ubT_h.at[0].at[uidx_v.at[isl]], ub_v.at[isl]))
            cps.append((ibT_h.at[0].at[iidx_v.at[isl]], ib_v.at[isl]))
            cps.append((utbT_h.at[0].at[uidx_v.at[isl]], utb0_v.at[isl]))
            cps.append((utbT_h.at[1].at[uidx_v.at[isl]], utb1_v.at[isl]))
            cps.append((utbT_h.at[2].at[uidx_v.at[isl]], utb2_v.at[isl]))
        return cps

    def fire(k, carry):
        for src, dst in chunk_copies(k):
            pltpu.async_copy(src, dst, sem)
        return carry

    lax.fori_loop(0, KF, fire, 0)
    for src, dst in bias_copies():
        pltpu.async_copy(src, dst, sem)

    def drain(k, carry):
        for src, dst in chunk_copies(k):
            pltpu.make_async_copy(src, dst, sem).wait()
        return carry

    lax.fori_loop(0, KF, drain, 0)
    for src, dst in bias_copies():
        pltpu.make_async_copy(src, dst, sem).wait()

    def comp(g, carry):
        sl = pl.ds(g * 16, 16)
        acc = U_v[0, sl] * I_v[0, sl]
        for k in range(1, KF):
            acc = acc + U_v[k, sl] * I_v[k, sl]
        d = didx_v[sl]
        utb = jnp.where(d == 0, utb0_v[sl],
                        jnp.where(d == 1, utb1_v[sl], utb2_v[sl]))
        tt = plsc.load_gather(t_v, [combo_v[sl]])
        res_v[sl] = acc + ub_v[sl] + ib_v[sl] + utb + tt
        return carry

    lax.fori_loop(0, BPW // 16, comp, 0)
    pltpu.sync_copy(res_v, out_h.at[pl.ds(base, BPW)])


_sc_call = functools.partial(
    pl.kernel,
    out_type=jax.ShapeDtypeStruct((B,), jnp.float32),
    mesh=plsc.VectorSubcoreMesh(core_axis_name="c", subcore_axis_name="s"),
    compiler_params=pltpu.CompilerParams(needs_layout_passes=False,
                                         use_tc_tiling_on_sc=False),
    scratch_types=[
        pltpu.VMEM((BPW,), jnp.int32),
        pltpu.VMEM((BPW,), jnp.int32),
        pltpu.VMEM((BPW,), jnp.int32),
        pltpu.VMEM((BPW,), jnp.int32),
        pltpu.VMEM((BPW,), jnp.int32),
        pltpu.VMEM((BPW,), jnp.int32),
        pltpu.VMEM((KF, BPW), jnp.float32),
        pltpu.VMEM((KF, BPW), jnp.float32),
        pltpu.VMEM((BPW,), jnp.float32),
        pltpu.VMEM((BPW,), jnp.float32),
        pltpu.VMEM((BPW,), jnp.float32),
        pltpu.VMEM((BPW,), jnp.float32),
        pltpu.VMEM((BPW,), jnp.float32),
        pltpu.VMEM((128,), jnp.float32),
        pltpu.VMEM((BPW,), jnp.float32),
        pltpu.SemaphoreType.DMA,
    ],
)(_sc_body)


def kernel(user_input, item_input, daytime_input, weekend_input, year_input,
           user_emb, item_emb, user_bias_w, item_bias_w,
           daytime_emb, weekend_emb, year_emb,
           daytime_bias_w, weekend_bias_w, year_bias_w,
           user_time_bias_w, fc_w1, fc_b1, fc_w2, fc_b2, global_bias):
    t128 = _build_time_table(daytime_emb, weekend_emb, year_emb,
                             daytime_bias_w, weekend_bias_w, year_bias_w,
                             fc_w1, fc_b1, fc_w2, fc_b2, global_bias)
    return _sc_call(
        user_input,
        item_input,
        daytime_input,
        weekend_input,
        year_input,
        user_emb.T,
        item_emb.T,
        user_bias_w.T,
        item_bias_w.T,
        user_time_bias_w.T,
        t128,
    )


# speedup vs baseline: 2.5484x; 2.5484x over previous
import functools

import jax
import jax.numpy as jnp
from jax import lax
from jax.experimental import pallas as pl
from jax.experimental.pallas import tpu as pltpu
from jax.experimental.pallas import tpu_sc as plsc

B = 16384
KF = 64
NC = 2
NS = 16
NW = NC * NS
BPW = B // NW
NCH = 4
CH = BPW // NCH


def _time_table_kernel(dt_ref, wk_ref, yr_ref, w1_ref, b1_ref, w2_ref,
                       b2_ref, gb_ref, out_ref):
    row = lax.broadcasted_iota(jnp.int32, (128, 128), 0)
    col = lax.broadcasted_iota(jnp.int32, (128, 128), 1)
    d = row // 40
    w = (row % 40) // 20
    y = row % 20
    ohd = jnp.where(col == d, 1.0, 0.0)
    ohw = jnp.where(col == w, 1.0, 0.0)
    ohy = jnp.where(col == y, 1.0, 0.0)
    f = (jnp.dot(ohd, dt_ref[...], preferred_element_type=jnp.float32)
         + jnp.dot(ohw, wk_ref[...], preferred_element_type=jnp.float32)
         + jnp.dot(ohy, yr_ref[...], preferred_element_type=jnp.float32))
    h = jnp.maximum(
        jnp.dot(f, w1_ref[...], preferred_element_type=jnp.float32)
        + b1_ref[...], 0.0)
    e = jnp.dot(h, w2_ref[...], preferred_element_type=jnp.float32)
    bias = f[:, 30:31] + f[:, 31:32] + f[:, 32:33]
    out_ref[...] = e + bias + b2_ref[0, 0] + gb_ref[0, 0]


def _build_time_table(daytime_emb, weekend_emb, year_emb,
                      daytime_bias_w, weekend_bias_w, year_bias_w,
                      fc_w1, fc_b1, fc_w2, fc_b2, global_bias):
    f32 = jnp.float32
    z = jnp.zeros((128, 128), f32)
    dt = z.at[0:3, 0:10].set(daytime_emb).at[0:3, 30].set(daytime_bias_w[:, 0])
    wk = z.at[0:2, 10:20].set(weekend_emb).at[0:2, 31].set(weekend_bias_w[:, 0])
    yr = z.at[0:20, 20:30].set(year_emb).at[0:20, 32].set(year_bias_w[:, 0])
    w1 = z.at[0:30, 0:10].set(fc_w1.T)
    w2 = z.at[0:10, 0:1].set(fc_w2.T)
    b1 = jnp.zeros((1, 128), f32).at[0, 0:10].set(fc_b1)
    tmat = pl.pallas_call(
        _time_table_kernel,
        out_shape=jax.ShapeDtypeStruct((128, 128), f32),
        in_specs=[
            pl.BlockSpec(memory_space=pltpu.VMEM),
            pl.BlockSpec(memory_space=pltpu.VMEM),
            pl.BlockSpec(memory_space=pltpu.VMEM),
            pl.BlockSpec(memory_space=pltpu.VMEM),
            pl.BlockSpec(memory_space=pltpu.VMEM),
            pl.BlockSpec(memory_space=pltpu.VMEM),
            pl.BlockSpec(memory_space=pltpu.SMEM),
            pl.BlockSpec(memory_space=pltpu.SMEM),
        ],
        out_specs=pl.BlockSpec(memory_space=pltpu.VMEM),
    )(dt, wk, yr, w1, b1, w2,
      fc_b2.reshape(1, 1), global_bias.reshape(1, 1))
    return tmat[:, 0]


def _sc_body(uin_h, iin_h, din_h, win_h, yin_h, uemb_h, iemb_h, ub_h,
             ib_h, utb_h, t_h, out_h,
             uidx_v, iidx_v, didx_v, widx_v, yidx_v, combo_v, u3_v,
             ue_v, ie_v, ub_v, ib_v, utb_v, t_v, res_v, sem):
    wid = lax.axis_index("s") * NC + lax.axis_index("c")
    base = wid * BPW
    pltpu.sync_copy(uin_h.at[pl.ds(base, BPW)], uidx_v)
    pltpu.sync_copy(iin_h.at[pl.ds(base, BPW)], iidx_v)
    pltpu.sync_copy(din_h.at[pl.ds(base, BPW)], didx_v)
    pltpu.sync_copy(win_h.at[pl.ds(base, BPW)], widx_v)
    pltpu.sync_copy(yin_h.at[pl.ds(base, BPW)], yidx_v)
    pltpu.sync_copy(t_h, t_v)

    for g in range(BPW // 16):
        sl = pl.ds(g * 16, 16)
        dd = didx_v[sl]
        combo_v[sl] = dd * 40 + widx_v[sl] * 20 + yidx_v[sl]
        u3_v[sl] = uidx_v[sl] * 3 + dd

    copies = []
    for j in range(NCH):
        isl = pl.ds(j * CH, CH)
        copies.append(pltpu.async_copy(
            uemb_h.at[uidx_v.at[isl]], ue_v.at[isl], sem))
        copies.append(pltpu.async_copy(
            iemb_h.at[iidx_v.at[isl]], ie_v.at[isl], sem))
        copies.append(pltpu.async_copy(
            ub_h.at[uidx_v.at[isl]], ub_v.at[isl], sem))
        copies.append(pltpu.async_copy(
            ib_h.at[iidx_v.at[isl]], ib_v.at[isl], sem))
        copies.append(pltpu.async_copy(
            utb_h.at[u3_v.at[isl]], utb_v.at[isl], sem))
    for c in copies:
        c.wait()

    lanes = lax.iota(jnp.int32, 16)

    def comp(g, carry):
        base16 = g * 16
        sl = pl.ds(base16, 16)
        acc = jnp.zeros((16,), jnp.float32)
        for u in range(16):
            r = base16 + u
            v = ue_v[r, pl.ds(0, 16)] * ie_v[r, pl.ds(0, 16)]
            for c in range(1, KF // 16):
                v = v + ue_v[r, pl.ds(c * 16, 16)] * ie_v[r, pl.ds(c * 16, 16)]
            acc = jnp.where(lanes == u, jnp.sum(v), acc)
        tt = plsc.load_gather(t_v, [combo_v[sl]])
        res_v[sl] = acc + ub_v[sl] + ib_v[sl] + utb_v[sl] + tt
        return carry

    lax.fori_loop(0, BPW // 16, comp, 0)
    pltpu.sync_copy(res_v, out_h.at[pl.ds(base, BPW)])


_sc_call = functools.partial(
    pl.kernel,
    out_type=jax.ShapeDtypeStruct((B,), jnp.float32),
    mesh=plsc.VectorSubcoreMesh(core_axis_name="c", subcore_axis_name="s"),
    compiler_params=pltpu.CompilerParams(needs_layout_passes=False,
                                         use_tc_tiling_on_sc=False),
    scratch_types=[
        pltpu.VMEM((BPW,), jnp.int32),
        pltpu.VMEM((BPW,), jnp.int32),
        pltpu.VMEM((BPW,), jnp.int32),
        pltpu.VMEM((BPW,), jnp.int32),
        pltpu.VMEM((BPW,), jnp.int32),
        pltpu.VMEM((BPW,), jnp.int32),
        pltpu.VMEM((BPW,), jnp.int32),
        pltpu.VMEM((BPW, KF), jnp.float32),
        pltpu.VMEM((BPW, KF), jnp.float32),
        pltpu.VMEM((BPW,), jnp.float32),
        pltpu.VMEM((BPW,), jnp.float32),
        pltpu.VMEM((BPW,), jnp.float32),
        pltpu.VMEM((128,), jnp.float32),
        pltpu.VMEM((BPW,), jnp.float32),
        pltpu.SemaphoreType.DMA,
    ],
)(_sc_body)


def kernel(user_input, item_input, daytime_input, weekend_input, year_input,
           user_emb, item_emb, user_bias_w, item_bias_w,
           daytime_emb, weekend_emb, year_emb,
           daytime_bias_w, weekend_bias_w, year_bias_w,
           user_time_bias_w, fc_w1, fc_b1, fc_w2, fc_b2, global_bias):
    t128 = _build_time_table(daytime_emb, weekend_emb, year_emb,
                             daytime_bias_w, weekend_bias_w, year_bias_w,
                             fc_w1, fc_b1, fc_w2, fc_b2, global_bias)
    return _sc_call(
        user_input,
        item_input,
        daytime_input,
        weekend_input,
        year_input,
        user_emb,
        item_emb,
        user_bias_w.reshape(-1),
        item_bias_w.reshape(-1),
        user_time_bias_w.reshape(-1),
        t128,
    )


# speedup vs baseline: 9.4553x; 3.7103x over previous
import functools

import jax
import jax.numpy as jnp
from jax import lax
from jax.experimental import pallas as pl
from jax.experimental.pallas import tpu as pltpu
from jax.experimental.pallas import tpu_sc as plsc

B = 16384
KF = 64
NC = 2
NS = 16
NW = NC * NS
BPW = B // NW
NCH = 4
CH = BPW // NCH


def _time_table_kernel(dt_ref, wk_ref, yr_ref, w1_ref, b1_ref, w2_ref,
                       b2_ref, gb_ref, out_ref):
    row = lax.broadcasted_iota(jnp.int32, (128, 128), 0)
    col = lax.broadcasted_iota(jnp.int32, (128, 128), 1)
    d = row // 40
    w = (row % 40) // 20
    y = row % 20
    ohd = jnp.where(col == d, 1.0, 0.0)
    ohw = jnp.where(col == w, 1.0, 0.0)
    ohy = jnp.where(col == y, 1.0, 0.0)
    f = (jnp.dot(ohd, dt_ref[...], preferred_element_type=jnp.float32)
         + jnp.dot(ohw, wk_ref[...], preferred_element_type=jnp.float32)
         + jnp.dot(ohy, yr_ref[...], preferred_element_type=jnp.float32))
    h = jnp.maximum(
        jnp.dot(f, w1_ref[...], preferred_element_type=jnp.float32)
        + b1_ref[...], 0.0)
    e = jnp.dot(h, w2_ref[...], preferred_element_type=jnp.float32)
    bias = f[:, 30:31] + f[:, 31:32] + f[:, 32:33]
    out_ref[...] = e + bias + b2_ref[0, 0] + gb_ref[0, 0]


def _build_time_table(daytime_emb, weekend_emb, year_emb,
                      daytime_bias_w, weekend_bias_w, year_bias_w,
                      fc_w1, fc_b1, fc_w2, fc_b2, global_bias):
    f32 = jnp.float32
    z = jnp.zeros((128, 128), f32)
    dt = z.at[0:3, 0:10].set(daytime_emb).at[0:3, 30].set(daytime_bias_w[:, 0])
    wk = z.at[0:2, 10:20].set(weekend_emb).at[0:2, 31].set(weekend_bias_w[:, 0])
    yr = z.at[0:20, 20:30].set(year_emb).at[0:20, 32].set(year_bias_w[:, 0])
    w1 = z.at[0:30, 0:10].set(fc_w1.T)
    w2 = z.at[0:10, 0:1].set(fc_w2.T)
    b1 = jnp.zeros((1, 128), f32).at[0, 0:10].set(fc_b1)
    tmat = pl.pallas_call(
        _time_table_kernel,
        out_shape=jax.ShapeDtypeStruct((128, 128), f32),
        in_specs=[
            pl.BlockSpec(memory_space=pltpu.VMEM),
            pl.BlockSpec(memory_space=pltpu.VMEM),
            pl.BlockSpec(memory_space=pltpu.VMEM),
            pl.BlockSpec(memory_space=pltpu.VMEM),
            pl.BlockSpec(memory_space=pltpu.VMEM),
            pl.BlockSpec(memory_space=pltpu.VMEM),
            pl.BlockSpec(memory_space=pltpu.SMEM),
            pl.BlockSpec(memory_space=pltpu.SMEM),
        ],
        out_specs=pl.BlockSpec(memory_space=pltpu.VMEM),
    )(dt, wk, yr, w1, b1, w2,
      fc_b2.reshape(1, 1), global_bias.reshape(1, 1))
    return tmat[:, 0]


def _sc_body(uin_h, iin_h, din_h, win_h, yin_h, uemb_h, iemb_h, ub_h,
             ib_h, utbT_h, t_h, out_h,
             uidx_v, iidx_v, didx_v, widx_v, yidx_v, combo_v,
             ue_v, ie_v, ub_v, ib_v, utb0_v, utb1_v, utb2_v, t_v, res_v, sem):
    wid = lax.axis_index("s") * NC + lax.axis_index("c")
    base = wid * BPW
    pltpu.sync_copy(uin_h.at[pl.ds(base, BPW)], uidx_v)
    pltpu.sync_copy(iin_h.at[pl.ds(base, BPW)], iidx_v)
    pltpu.sync_copy(din_h.at[pl.ds(base, BPW)], didx_v)
    pltpu.sync_copy(win_h.at[pl.ds(base, BPW)], widx_v)
    pltpu.sync_copy(yin_h.at[pl.ds(base, BPW)], yidx_v)
    pltpu.sync_copy(t_h, t_v)

    for g in range(BPW // 16):
        sl = pl.ds(g * 16, 16)
        combo_v[sl] = didx_v[sl] * 40 + widx_v[sl] * 20 + yidx_v[sl]

    copies = []
    for j in range(NCH):
        isl = pl.ds(j * CH, CH)
        copies.append(pltpu.async_copy(
            uemb_h.at[uidx_v.at[isl]], ue_v.at[isl], sem))
        copies.append(pltpu.async_copy(
            iemb_h.at[iidx_v.at[isl]], ie_v.at[isl], sem))
        copies.append(pltpu.async_copy(
            ub_h.at[0].at[uidx_v.at[isl]], ub_v.at[isl], sem))
        copies.append(pltpu.async_copy(
            ib_h.at[0].at[iidx_v.at[isl]], ib_v.at[isl], sem))
        copies.append(pltpu.async_copy(
            utbT_h.at[0].at[uidx_v.at[isl]], utb0_v.at[isl], sem))
        copies.append(pltpu.async_copy(
            utbT_h.at[1].at[uidx_v.at[isl]], utb1_v.at[isl], sem))
        copies.append(pltpu.async_copy(
            utbT_h.at[2].at[uidx_v.at[isl]], utb2_v.at[isl], sem))
    for c in copies:
        c.wait()

    lanes = lax.iota(jnp.int32, 16)

    def comp(g, carry):
        base16 = g * 16
        sl = pl.ds(base16, 16)
        acc = jnp.zeros((16,), jnp.float32)
        for u in range(16):
            r = base16 + u
            v = ue_v[r, pl.ds(0, 16)] * ie_v[r, pl.ds(0, 16)]
            for c in range(1, KF // 16):
                v = v + ue_v[r, pl.ds(c * 16, 16)] * ie_v[r, pl.ds(c * 16, 16)]
            acc = jnp.where(lanes == u, jnp.sum(v), acc)
        tt = plsc.load_gather(t_v, [combo_v[sl]])
        d = didx_v[sl]
        utbv = jnp.where(d == 0, utb0_v[sl],
                         jnp.where(d == 1, utb1_v[sl], utb2_v[sl]))
        res_v[sl] = acc + ub_v[sl] + ib_v[sl] + utbv + tt
        return carry

    lax.fori_loop(0, BPW // 16, comp, 0)
    pltpu.sync_copy(res_v, out_h.at[pl.ds(base, BPW)])


_sc_call = functools.partial(
    pl.kernel,
    out_type=jax.ShapeDtypeStruct((B,), jnp.float32),
    mesh=plsc.VectorSubcoreMesh(core_axis_name="c", subcore_axis_name="s"),
    compiler_params=pltpu.CompilerParams(needs_layout_passes=False,
                                         use_tc_tiling_on_sc=False),
    scratch_types=[
        pltpu.VMEM((BPW,), jnp.int32),
        pltpu.VMEM((BPW,), jnp.int32),
        pltpu.VMEM((BPW,), jnp.int32),
        pltpu.VMEM((BPW,), jnp.int32),
        pltpu.VMEM((BPW,), jnp.int32),
        pltpu.VMEM((BPW,), jnp.int32),
        pltpu.VMEM((BPW, KF), jnp.float32),
        pltpu.VMEM((BPW, KF), jnp.float32),
        pltpu.VMEM((BPW,), jnp.float32),
        pltpu.VMEM((BPW,), jnp.float32),
        pltpu.VMEM((BPW,), jnp.float32),
        pltpu.VMEM((BPW,), jnp.float32),
        pltpu.VMEM((BPW,), jnp.float32),
        pltpu.VMEM((128,), jnp.float32),
        pltpu.VMEM((BPW,), jnp.float32),
        pltpu.SemaphoreType.DMA,
    ],
)(_sc_body)


def kernel(user_input, item_input, daytime_input, weekend_input, year_input,
           user_emb, item_emb, user_bias_w, item_bias_w,
           daytime_emb, weekend_emb, year_emb,
           daytime_bias_w, weekend_bias_w, year_bias_w,
           user_time_bias_w, fc_w1, fc_b1, fc_w2, fc_b2, global_bias):
    t128 = _build_time_table(daytime_emb, weekend_emb, year_emb,
                             daytime_bias_w, weekend_bias_w, year_bias_w,
                             fc_w1, fc_b1, fc_w2, fc_b2, global_bias)
    return _sc_call(
        user_input,
        item_input,
        daytime_input,
        weekend_input,
        year_input,
        user_emb,
        item_emb,
        user_bias_w.T,
        item_bias_w.T,
        user_time_bias_w.T,
        t128,
    )


# speedup vs baseline: 11.6851x; 1.2358x over previous
import functools

import jax
import jax.numpy as jnp
from jax import lax
from jax.experimental import pallas as pl
from jax.experimental.pallas import tpu as pltpu
from jax.experimental.pallas import tpu_sc as plsc

B = 16384
KF = 64
NC = 2
NS = 16
NW = NC * NS
BPW = B // NW
NCH = 4
CH = BPW // NCH
MCH = 32


def _time_table_kernel(dt_ref, wk_ref, yr_ref, w1_ref, b1_ref, w2_ref,
                       b2_ref, gb_ref, out_ref):
    row = lax.broadcasted_iota(jnp.int32, (128, 128), 0)
    col = lax.broadcasted_iota(jnp.int32, (128, 128), 1)
    d = row // 40
    w = (row % 40) // 20
    y = row % 20
    ohd = jnp.where(col == d, 1.0, 0.0)
    ohw = jnp.where(col == w, 1.0, 0.0)
    ohy = jnp.where(col == y, 1.0, 0.0)
    f = (jnp.dot(ohd, dt_ref[...], preferred_element_type=jnp.float32)
         + jnp.dot(ohw, wk_ref[...], preferred_element_type=jnp.float32)
         + jnp.dot(ohy, yr_ref[...], preferred_element_type=jnp.float32))
    h = jnp.maximum(
        jnp.dot(f, w1_ref[...], preferred_element_type=jnp.float32)
        + b1_ref[...], 0.0)
    e = jnp.dot(h, w2_ref[...], preferred_element_type=jnp.float32)
    bias = f[:, 30:31] + f[:, 31:32] + f[:, 32:33]
    out_ref[...] = e + bias + b2_ref[0, 0] + gb_ref[0, 0]


def _build_time_table(daytime_emb, weekend_emb, year_emb,
                      daytime_bias_w, weekend_bias_w, year_bias_w,
                      fc_w1, fc_b1, fc_w2, fc_b2, global_bias):
    f32 = jnp.float32
    z = jnp.zeros((128, 128), f32)
    dt = z.at[0:3, 0:10].set(daytime_emb).at[0:3, 30].set(daytime_bias_w[:, 0])
    wk = z.at[0:2, 10:20].set(weekend_emb).at[0:2, 31].set(weekend_bias_w[:, 0])
    yr = z.at[0:20, 20:30].set(year_emb).at[0:20, 32].set(year_bias_w[:, 0])
    w1 = z.at[0:30, 0:10].set(fc_w1.T)
    w2 = z.at[0:10, 0:1].set(fc_w2.T)
    b1 = jnp.zeros((1, 128), f32).at[0, 0:10].set(fc_b1)
    tmat = pl.pallas_call(
        _time_table_kernel,
        out_shape=jax.ShapeDtypeStruct((128, 128), f32),
        in_specs=[
            pl.BlockSpec(memory_space=pltpu.VMEM),
            pl.BlockSpec(memory_space=pltpu.VMEM),
            pl.BlockSpec(memory_space=pltpu.VMEM),
            pl.BlockSpec(memory_space=pltpu.VMEM),
            pl.BlockSpec(memory_space=pltpu.VMEM),
            pl.BlockSpec(memory_space=pltpu.VMEM),
            pl.BlockSpec(memory_space=pltpu.SMEM),
            pl.BlockSpec(memory_space=pltpu.SMEM),
        ],
        out_specs=pl.BlockSpec(memory_space=pltpu.VMEM),
    )(dt, wk, yr, w1, b1, w2,
      fc_b2.reshape(1, 1), global_bias.reshape(1, 1))
    return tmat[:, 0]


def _bias_body(uin_h, iin_h, din_h, win_h, yin_h, ub_h, ib_h, utbT_h, t_h,
               out_h,
               uidx_v, iidx_v, didx_v, widx_v, yidx_v, combo_v,
               ub_v, ib_v, utb0_v, utb1_v, utb2_v, t_v, res_v, sem):
    wid = lax.axis_index("s") * NC + lax.axis_index("c")
    base = wid * BPW
    pltpu.sync_copy(uin_h.at[pl.ds(base, BPW)], uidx_v)
    pltpu.sync_copy(iin_h.at[pl.ds(base, BPW)], iidx_v)
    pltpu.sync_copy(din_h.at[pl.ds(base, BPW)], didx_v)
    pltpu.sync_copy(win_h.at[pl.ds(base, BPW)], widx_v)
    pltpu.sync_copy(yin_h.at[pl.ds(base, BPW)], yidx_v)
    pltpu.sync_copy(t_h, t_v)

    for g in range(BPW // 16):
        sl = pl.ds(g * 16, 16)
        combo_v[sl] = didx_v[sl] * 40 + widx_v[sl] * 20 + yidx_v[sl]

    copies = []
    for j in range(NCH):
        isl = pl.ds(j * CH, CH)
        copies.append(pltpu.async_copy(
            ub_h.at[0].at[uidx_v.at[isl]], ub_v.at[isl], sem))
        copies.append(pltpu.async_copy(
            ib_h.at[0].at[iidx_v.at[isl]], ib_v.at[isl], sem))
        copies.append(pltpu.async_copy(
            utbT_h.at[0].at[uidx_v.at[isl]], utb0_v.at[isl], sem))
        copies.append(pltpu.async_copy(
            utbT_h.at[1].at[uidx_v.at[isl]], utb1_v.at[isl], sem))
        copies.append(pltpu.async_copy(
            utbT_h.at[2].at[uidx_v.at[isl]], utb2_v.at[isl], sem))
    for c in copies:
        c.wait()

    def comp(g, carry):
        sl = pl.ds(g * 16, 16)
        tt = plsc.load_gather(t_v, [combo_v[sl]])
        d = didx_v[sl]
        utbv = jnp.where(d == 0, utb0_v[sl],
                         jnp.where(d == 1, utb1_v[sl], utb2_v[sl]))
        res_v[sl] = ub_v[sl] + ib_v[sl] + utbv + tt
        return carry

    lax.fori_loop(0, BPW // 16, comp, 0)
    pltpu.sync_copy(res_v, out_h.at[pl.ds(base, BPW)])


_bias_call = functools.partial(
    pl.kernel,
    out_type=jax.ShapeDtypeStruct((B,), jnp.float32),
    mesh=plsc.VectorSubcoreMesh(core_axis_name="c", subcore_axis_name="s"),
    compiler_params=pltpu.CompilerParams(needs_layout_passes=False,
                                         use_tc_tiling_on_sc=False),
    scratch_types=[
        pltpu.VMEM((BPW,), jnp.int32),
        pltpu.VMEM((BPW,), jnp.int32),
        pltpu.VMEM((BPW,), jnp.int32),
        pltpu.VMEM((BPW,), jnp.int32),
        pltpu.VMEM((BPW,), jnp.int32),
        pltpu.VMEM((BPW,), jnp.int32),
        pltpu.VMEM((BPW,), jnp.float32),
        pltpu.VMEM((BPW,), jnp.float32),
        pltpu.VMEM((BPW,), jnp.float32),
        pltpu.VMEM((BPW,), jnp.float32),
        pltpu.VMEM((BPW,), jnp.float32),
        pltpu.VMEM((128,), jnp.float32),
        pltpu.VMEM((BPW,), jnp.float32),
        pltpu.SemaphoreType.DMA,
    ],
)(_bias_body)


def _main_body(uin_h, iin_h, bias_h, uemb_h, iemb_h, out_h,
               uidx_v, iidx_v, bias_v, blku_v, blki_v, res_v, sem):
    wid = lax.axis_index("s") * NC + lax.axis_index("c")
    base = wid * BPW
    pltpu.sync_copy(uin_h.at[pl.ds(base, BPW)], uidx_v)
    pltpu.sync_copy(iin_h.at[pl.ds(base, BPW)], iidx_v)
    pltpu.sync_copy(bias_h.at[pl.ds(base, BPW)], bias_v)

    lanes = lax.iota(jnp.int32, 16)

    def chunk(cix, carry):
        cb = cix * MCH

        def fire(g, c2):
            sl = pl.ds(cb + g * 16, 16)
            vu = uidx_v[sl]
            vi = iidx_v[sl]
            for j in range(16):
                u = vu[j]
                u8 = pl.multiple_of(u - lax.rem(u, 8), 8)
                pltpu.async_copy(uemb_h.at[pl.ds(u8, 8)],
                                 blku_v.at[g * 16 + j], sem)
                i = vi[j]
                i8 = pl.multiple_of(i - lax.rem(i, 8), 8)
                pltpu.async_copy(iemb_h.at[pl.ds(i8, 8)],
                                 blki_v.at[g * 16 + j], sem)
            return c2

        lax.fori_loop(0, MCH // 16, fire, 0)

        def drain(r, c2):
            pltpu.make_async_copy(uemb_h.at[pl.ds(0, 8)],
                                  blku_v.at[r], sem).wait()
            pltpu.make_async_copy(iemb_h.at[pl.ds(0, 8)],
                                  blki_v.at[r], sem).wait()
            return c2

        lax.fori_loop(0, MCH, drain, 0)

        def comp(g, c2):
            sl = pl.ds(cb + g * 16, 16)
            vu = uidx_v[sl]
            vi = iidx_v[sl]
            acc = jnp.zeros((16,), jnp.float32)
            for j in range(16):
                r = g * 16 + j
                uu = lax.rem(vu[j], 8)
                ii = lax.rem(vi[j], 8)
                v = (blku_v[r, uu, pl.ds(0, 16)]
                     * blki_v[r, ii, pl.ds(0, 16)])
                for c in range(1, KF // 16):
                    v = v + (blku_v[r, uu, pl.ds(c * 16, 16)]
                             * blki_v[r, ii, pl.ds(c * 16, 16)])
                acc = jnp.where(lanes == j, jnp.sum(v), acc)
            res_v[sl] = acc + bias_v[sl]
            return c2

        lax.fori_loop(0, MCH // 16, comp, 0)
        return carry

    lax.fori_loop(0, BPW // MCH, chunk, 0)
    pltpu.sync_copy(res_v, out_h.at[pl.ds(base, BPW)])


_main_call = functools.partial(
    pl.kernel,
    out_type=jax.ShapeDtypeStruct((B,), jnp.float32),
    mesh=plsc.VectorSubcoreMesh(core_axis_name="c", subcore_axis_name="s"),
    compiler_params=pltpu.CompilerParams(needs_layout_passes=False),
    scratch_types=[
        pltpu.VMEM((BPW,), jnp.int32),
        pltpu.VMEM((BPW,), jnp.int32),
        pltpu.VMEM((BPW,), jnp.float32),
        pltpu.VMEM((MCH, 8, KF), jnp.float32),
        pltpu.VMEM((MCH, 8, KF), jnp.float32),
        pltpu.VMEM((BPW,), jnp.float32),
        pltpu.SemaphoreType.DMA,
    ],
)(_main_body)


def kernel(user_input, item_input, daytime_input, weekend_input, year_input,
           user_emb, item_emb, user_bias_w, item_bias_w,
           daytime_emb, weekend_emb, year_emb,
           daytime_bias_w, weekend_bias_w, year_bias_w,
           user_time_bias_w, fc_w1, fc_b1, fc_w2, fc_b2, global_bias):
    t128 = _build_time_table(daytime_emb, weekend_emb, year_emb,
                             daytime_bias_w, weekend_bias_w, year_bias_w,
                             fc_w1, fc_b1, fc_w2, fc_b2, global_bias)
    bias = _bias_call(
        user_input,
        item_input,
        daytime_input,
        weekend_input,
        year_input,
        user_bias_w.T,
        item_bias_w.T,
        user_time_bias_w.T,
        t128,
    )
    return _main_call(
        user_input,
        item_input,
        bias,
        user_emb,
        item_emb,
    )
